# baseline (speedup 1.0000x reference)
import functools

import jax
import jax.numpy as jnp
from jax import lax
from jax.experimental import pallas as pl
from jax.experimental.pallas import tpu as pltpu
from jax.experimental.pallas import tpu_sc as plsc

NC = 2
NS = 16
CHUNK = 128
TC_BLK = 2048
N_PAD = 1 << 20
N_LINES = N_PAD // 4
Q_SHIFT = 18


def _dequant_body(w0, w1, w2, w3, s_ref, o_ref):
    y = jnp.concatenate([w.astype(jnp.float32) for w in
                         (w0[...], w1[...], w2[...], w3[...])], axis=1)
    o_ref[...] = y * s_ref[...]


@functools.cache
def _build_dequant(emb_dim):
    bpq = N_LINES // TC_BLK
    return pl.pallas_call(
        _dequant_body,
        grid=(bpq,),
        in_specs=[
            pl.BlockSpec((TC_BLK, emb_dim),
                         lambda i, q=q: (q * bpq + i, 0))
            for q in range(4)
        ] + [pl.BlockSpec((1, 4 * emb_dim), lambda i: (0, 0))],
        out_specs=pl.BlockSpec((TC_BLK, 4 * emb_dim), lambda i: (i, 0)),
        out_shape=jax.ShapeDtypeStruct((N_LINES, 4 * emb_dim), jnp.float32),
    )


@functools.cache
def _build_lookup(n_rows, emb_dim, n_chunks_per_tile):
    nw = NC * NS
    lanes = 4 * emb_dim
    mesh = plsc.VectorSubcoreMesh(core_axis_name="c", subcore_axis_name="s")

    @functools.partial(
        pl.kernel,
        mesh=mesh,
        out_type=jax.ShapeDtypeStruct((n_rows // 4, lanes), jnp.float32),
        scratch_types=[
            pltpu.VMEM((n_chunks_per_tile, CHUNK), jnp.int32),
            pltpu.VMEM((CHUNK,), jnp.int32),
            pltpu.VMEM((CHUNK, lanes), jnp.float32),
            pltpu.VMEM((CHUNK // 4, lanes), jnp.float32),
            pltpu.SemaphoreType.DMA,
        ],
        compiler_params=pltpu.CompilerParams(needs_layout_passes=False,
                                             use_tc_tiling_on_sc=False),
    )
    def body(idx_hbm, table_hbm, out_hbm, idx_v, line_v, win_v, out_v, sem):
        wid = lax.axis_index("s") * NC + lax.axis_index("c")
        pltpu.sync_copy(
            idx_hbm.at[pl.ds(wid * n_chunks_per_tile, n_chunks_per_tile)],
            idx_v)

        lane = lax.iota(jnp.int32, 16)
        line_pat = lane // 4
        col_pat = 32 * (lane % 4)
        tile_base = wid * (n_chunks_per_tile * (CHUNK // 4))

        def do_chunk(j, _):
            def prep(g, _):
                ii = idx_v[j, pl.ds(16 * g, 16)]
                line_v[pl.ds(16 * g, 16)] = ii & (N_LINES - 1)
                return 0

            lax.fori_loop(0, CHUNK // 16, prep, 0)
            pltpu.async_copy(table_hbm.at[line_v], win_v, sem).wait()

            def select(g, _):
                ii = idx_v[j, pl.ds(16 * g, 16)]
                col_base = 32 * (ii >> Q_SHIFT)
                rows = 16 * g + lane
                out_line = 4 * g + line_pat
                for c in range(32):
                    x = plsc.load_gather(win_v, [rows, col_base + c])
                    plsc.store_scatter(out_v, [out_line, col_pat + c], x)
                return 0

            lax.fori_loop(0, CHUNK // 16, select, 0)
            pltpu.sync_copy(
                out_v,
                out_hbm.at[pl.ds(tile_base + j * (CHUNK // 4), CHUNK // 4)])
            return 0

        lax.fori_loop(0, n_chunks_per_tile, do_chunk, 0)

    return body


def kernel(input, weight, weight_scale):
    batch, n_fields = input.shape
    n_rows = batch * n_fields
    n_emb, emb_dim = weight.shape
    nw = NC * NS
    n_chunks_per_tile = n_rows // (nw * CHUNK)

    idx = input.reshape(nw * n_chunks_per_tile, CHUNK)
    scale_tiled = jnp.tile(weight_scale, 4).reshape(1, 4 * emb_dim)
    w_pad = jnp.pad(weight, ((0, N_PAD - n_emb), (0, 0)))

    table_f32 = _build_dequant(emb_dim)(w_pad, w_pad, w_pad, w_pad,
                                        scale_tiled)
    out = _build_lookup(n_rows, emb_dim, n_chunks_per_tile)(idx, table_f32)
    return out.reshape(batch, n_fields, emb_dim)


# speedup vs baseline: 1.0331x; 1.0331x over previous
import functools

import jax
import jax.numpy as jnp
from jax import lax
from jax.experimental import pallas as pl
from jax.experimental.pallas import tpu as pltpu
from jax.experimental.pallas import tpu_sc as plsc

NC = 2
NS = 16
CHUNK = 128
TC_BLK = 2048
N_PAD = 1 << 20
N_LINES = N_PAD // 4
Q_SHIFT = 18


def _dequant_body(w0, w1, w2, w3, s_ref, o_ref):
    y = jnp.concatenate([w.astype(jnp.float32) for w in
                         (w0[...], w1[...], w2[...], w3[...])], axis=1)
    o_ref[...] = y * s_ref[...]


@functools.cache
def _build_dequant(n_emb, emb_dim):
    bpq = N_LINES // TC_BLK
    last_blk = (n_emb - 1) // TC_BLK
    return pl.pallas_call(
        _dequant_body,
        grid=(bpq,),
        in_specs=[
            pl.BlockSpec((TC_BLK, emb_dim),
                         lambda i, q=q: (jnp.minimum(q * bpq + i, last_blk), 0))
            for q in range(4)
        ] + [pl.BlockSpec((1, 4 * emb_dim), lambda i: (0, 0))],
        out_specs=pl.BlockSpec((TC_BLK, 4 * emb_dim), lambda i: (i, 0)),
        out_shape=jax.ShapeDtypeStruct((N_LINES, 4 * emb_dim), jnp.float32),
    )


@functools.cache
def _build_lookup(n_rows, emb_dim, n_chunks_per_tile):
    nw = NC * NS
    lanes = 4 * emb_dim
    olines = CHUNK // 4
    mesh = plsc.VectorSubcoreMesh(core_axis_name="c", subcore_axis_name="s")

    @functools.partial(
        pl.kernel,
        mesh=mesh,
        out_type=jax.ShapeDtypeStruct((n_rows // 4, lanes), jnp.float32),
        scratch_types=[
            pltpu.VMEM((n_chunks_per_tile, CHUNK), jnp.int32),
            pltpu.VMEM((n_chunks_per_tile, CHUNK), jnp.int32),
            pltpu.VMEM((CHUNK, lanes), jnp.float32),
            pltpu.VMEM((CHUNK, lanes), jnp.float32),
            pltpu.VMEM((olines, lanes), jnp.float32),
            pltpu.VMEM((olines, lanes), jnp.float32),
            pltpu.SemaphoreType.DMA,
            pltpu.SemaphoreType.DMA,
            pltpu.SemaphoreType.DMA,
            pltpu.SemaphoreType.DMA,
        ],
        compiler_params=pltpu.CompilerParams(needs_layout_passes=False,
                                             use_tc_tiling_on_sc=False),
    )
    def body(idx_hbm, table_hbm, out_hbm, idx_v, line_v, win0, win1,
             ob0, ob1, sg0, sg1, so0, so1):
        wid = lax.axis_index("s") * NC + lax.axis_index("c")
        pltpu.sync_copy(
            idx_hbm.at[pl.ds(wid * n_chunks_per_tile, n_chunks_per_tile)],
            idx_v)

        lane = lax.iota(jnp.int32, 16)
        line_pat = lane // 4
        col_pat = 32 * (lane % 4)
        tile_base = wid * (n_chunks_per_tile * olines)

        def prep(t, _):
            j, g = t // (CHUNK // 16), t % (CHUNK // 16)
            ii = idx_v[j, pl.ds(16 * g, 16)]
            line_v[j, pl.ds(16 * g, 16)] = ii & (N_LINES - 1)
            return 0

        lax.fori_loop(0, n_chunks_per_tile * (CHUNK // 16), prep, 0)

        def g_desc(j, buf, sem):
            return pltpu.make_async_copy(table_hbm.at[line_v.at[j]], buf, sem)

        def o_desc(j, buf, sem):
            return pltpu.make_async_copy(
                buf, out_hbm.at[pl.ds(tile_base + j * olines, olines)], sem)

        def select(j, win_v, out_v):
            def sel(g, _):
                ii = idx_v[j, pl.ds(16 * g, 16)]
                col_base = 32 * (ii >> Q_SHIFT)
                rows = 16 * g + lane
                out_line = 4 * g + line_pat
                for c in range(32):
                    x = plsc.load_gather(win_v, [rows, col_base + c])
                    plsc.store_scatter(out_v, [out_line, col_pat + c], x)
                return 0

            lax.fori_loop(0, CHUNK // 16, sel, 0)

        n_half = n_chunks_per_tile // 2
        g_desc(0, win0, sg0).start()

        def step(jj, _):
            j0 = 2 * jj
            j1 = j0 + 1
            g_desc(j1, win1, sg1).start()
            g_desc(j0, win0, sg0).wait()
            select(j0, win0, ob0)

            @pl.when(jj > 0)
            def _():
                o_desc(j0 - 2, ob0, so0).wait()

            o_desc(j0, ob0, so0).start()

            @pl.when(jj < n_half - 1)
            def _():
                g_desc(j0 + 2, win0, sg0).start()

            g_desc(j1, win1, sg1).wait()
            select(j1, win1, ob1)

            @pl.when(jj > 0)
            def _():
                o_desc(j1 - 2, ob1, so1).wait()

            o_desc(j1, ob1, so1).start()
            return 0

        lax.fori_loop(0, n_half, step, 0)
        o_desc(n_chunks_per_tile - 2, ob0, so0).wait()
        o_desc(n_chunks_per_tile - 1, ob1, so1).wait()

    return body


def kernel(input, weight, weight_scale):
    batch, n_fields = input.shape
    n_rows = batch * n_fields
    n_emb, emb_dim = weight.shape
    nw = NC * NS
    n_chunks_per_tile = n_rows // (nw * CHUNK)

    idx = input.reshape(nw * n_chunks_per_tile, CHUNK)
    scale_tiled = jnp.tile(weight_scale, 4).reshape(1, 4 * emb_dim)

    table_f32 = _build_dequant(n_emb, emb_dim)(weight, weight, weight,
                                               weight, scale_tiled)
    out = _build_lookup(n_rows, emb_dim, n_chunks_per_tile)(idx, table_f32)
    return out.reshape(batch, n_fields, emb_dim)


# speedup vs baseline: 1.1399x; 1.1034x over previous
import functools

import jax
import jax.numpy as jnp
from jax import lax
from jax.experimental import pallas as pl
from jax.experimental.pallas import tpu as pltpu
from jax.experimental.pallas import tpu_sc as plsc

NC = 2
NS = 16
CHUNK = 128
TC_BLK = 2048
N_PAD = 1 << 20
N_LINES = N_PAD // 4
Q_SHIFT = 18


def _dequant_body(w0, w1, w2, w3, s_ref, o_ref):
    y = jnp.concatenate([w.astype(jnp.float32) for w in
                         (w0[...], w1[...], w2[...], w3[...])], axis=1)
    o_ref[...] = y * s_ref[...]


@functools.cache
def _build_dequant(emb_dim):
    bpq = N_LINES // TC_BLK
    return pl.pallas_call(
        _dequant_body,
        grid=(bpq,),
        in_specs=[
            pl.BlockSpec((TC_BLK, emb_dim),
                         lambda i, q=q: (q * bpq + i, 0))
            for q in range(4)
        ] + [pl.BlockSpec((1, 4 * emb_dim), lambda i: (0, 0))],
        out_specs=pl.BlockSpec((TC_BLK, 4 * emb_dim), lambda i: (i, 0)),
        out_shape=jax.ShapeDtypeStruct((N_LINES, 4 * emb_dim), jnp.float32),
    )


@functools.cache
def _build_lookup(n_rows, emb_dim, n_chunks_per_tile):
    nw = NC * NS
    lanes = 4 * emb_dim
    gps = 2
    sc_rows = gps * CHUNK
    olines = sc_rows // 4
    n_super = n_chunks_per_tile // gps
    mesh = plsc.VectorSubcoreMesh(core_axis_name="c", subcore_axis_name="s")

    @functools.partial(
        pl.kernel,
        mesh=mesh,
        out_type=jax.ShapeDtypeStruct((n_rows // 4, lanes), jnp.float32),
        scratch_types=[
            pltpu.VMEM((n_chunks_per_tile, CHUNK), jnp.int32),
            pltpu.VMEM((n_chunks_per_tile, CHUNK), jnp.int32),
            pltpu.VMEM((sc_rows, lanes), jnp.float32),
            pltpu.VMEM((sc_rows, lanes), jnp.float32),
            pltpu.VMEM((olines, lanes), jnp.float32),
            pltpu.VMEM((olines, lanes), jnp.float32),
            pltpu.SemaphoreType.DMA,
            pltpu.SemaphoreType.DMA,
            pltpu.SemaphoreType.DMA,
            pltpu.SemaphoreType.DMA,
        ],
        compiler_params=pltpu.CompilerParams(needs_layout_passes=False,
                                             use_tc_tiling_on_sc=False),
    )
    def body(idx_hbm, table_hbm, out_hbm, idx_v, line_v, win0, win1,
             ob0, ob1, sg0, sg1, so0, so1):
        wid = lax.axis_index("s") * NC + lax.axis_index("c")
        pltpu.sync_copy(
            idx_hbm.at[pl.ds(wid * n_chunks_per_tile, n_chunks_per_tile)],
            idx_v)

        lane = lax.iota(jnp.int32, 16)
        line_pat = lane // 4
        col_pat = 32 * (lane % 4)
        tile_base = wid * (n_chunks_per_tile * (CHUNK // 4))

        def prep(t, _):
            j, g = t // (CHUNK // 16), t % (CHUNK // 16)
            ii = idx_v[j, pl.ds(16 * g, 16)]
            line_v[j, pl.ds(16 * g, 16)] = ii & (N_LINES - 1)
            return 0

        lax.fori_loop(0, n_chunks_per_tile * (CHUNK // 16), prep, 0)

        def g_start(s, buf, sem):
            for u in range(gps):
                pltpu.make_async_copy(
                    table_hbm.at[line_v.at[gps * s + u]],
                    buf.at[pl.ds(u * CHUNK, CHUNK)], sem).start()

        def g_wait(s, buf, sem):
            for u in range(gps):
                pltpu.make_async_copy(
                    table_hbm.at[line_v.at[gps * s + u]],
                    buf.at[pl.ds(u * CHUNK, CHUNK)], sem).wait()

        def o_desc(s, buf, sem):
            return pltpu.make_async_copy(
                buf, out_hbm.at[pl.ds(tile_base + s * olines, olines)], sem)

        def select(s, win_v, out_v):
            def sel(g, _):
                j = gps * s + g // (CHUNK // 16)
                gg = g % (CHUNK // 16)
                ii = idx_v[j, pl.ds(16 * gg, 16)]
                col_base = 32 * (ii >> Q_SHIFT)
                rows = 16 * g + lane
                out_line = 4 * g + line_pat
                for c in range(32):
                    x = plsc.load_gather(win_v, [rows, col_base + c])
                    plsc.store_scatter(out_v, [out_line, col_pat + c], x)
                return 0

            lax.fori_loop(0, sc_rows // 16, sel, 0)

        n_half = n_super // 2
        g_start(0, win0, sg0)

        def step(jj, _):
            s0 = 2 * jj
            s1 = s0 + 1
            g_start(s1, win1, sg1)
            g_wait(s0, win0, sg0)
            select(s0, win0, ob0)

            @pl.when(jj > 0)
            def _():
                o_desc(s0 - 2, ob0, so0).wait()

            o_desc(s0, ob0, so0).start()

            @pl.when(jj < n_half - 1)
            def _():
                g_start(s0 + 2, win0, sg0)

            g_wait(s1, win1, sg1)
            select(s1, win1, ob1)

            @pl.when(jj > 0)
            def _():
                o_desc(s1 - 2, ob1, so1).wait()

            o_desc(s1, ob1, so1).start()
            return 0

        lax.fori_loop(0, n_half, step, 0)
        o_desc(n_super - 2, ob0, so0).wait()
        o_desc(n_super - 1, ob1, so1).wait()

    return body


def kernel(input, weight, weight_scale):
    batch, n_fields = input.shape
    n_rows = batch * n_fields
    n_emb, emb_dim = weight.shape
    nw = NC * NS
    n_chunks_per_tile = n_rows // (nw * CHUNK)

    idx = input.reshape(nw * n_chunks_per_tile, CHUNK)
    scale_tiled = jnp.tile(weight_scale, 4).reshape(1, 4 * emb_dim)
    w_pad = jnp.pad(weight, ((0, N_PAD - n_emb), (0, 0)))

    table_f32 = _build_dequant(emb_dim)(w_pad, w_pad, w_pad, w_pad,
                                        scale_tiled)
    out = _build_lookup(n_rows, emb_dim, n_chunks_per_tile)(idx, table_f32)
    return out.reshape(batch, n_fields, emb_dim)
